# baseline (speedup 1.0000x reference)
import functools

import jax
import jax.numpy as jnp
from jax import lax
from jax.experimental import pallas as pl
from jax.experimental.pallas import tpu as pltpu
from jax.experimental.pallas import tpu_sc as plsc

_TOK_BLOCK = 2048
_N_WORKERS = 32


def _router_probs_kernel(x_ref, w_ref, b_ref, probs_ref):
    x = x_ref[...]
    logits = jax.lax.dot_general(
        x, w_ref[...], (((1,), (1,)), ((), ())),
        preferred_element_type=jnp.float32,
    )
    logits = logits + b_ref[...]
    m = jnp.max(logits, axis=1, keepdims=True)
    e = jnp.exp(logits - m)
    z = jnp.sum(e, axis=1, keepdims=True)
    probs_ref[...] = e / z


def _tc_probs(x, W, b):
    n_tok, d_model = x.shape
    n_exp = W.shape[0]
    t = _TOK_BLOCK
    return pl.pallas_call(
        _router_probs_kernel,
        grid=(n_tok // t,),
        in_specs=[
            pl.BlockSpec((t, d_model), lambda i: (i, 0)),
            pl.BlockSpec((n_exp, d_model), lambda i: (0, 0)),
            pl.BlockSpec((1, n_exp), lambda i: (0, 0)),
        ],
        out_specs=pl.BlockSpec((t, n_exp), lambda i: (i, 0)),
        out_shape=jax.ShapeDtypeStruct((n_tok, n_exp), jnp.float32),
        compiler_params=pltpu.CompilerParams(
            dimension_semantics=("parallel",),
        ),
    )(x, W.reshape(n_exp, d_model), b.reshape(1, n_exp))


def _sc_top2_body(rows_per_tile, n_exp, probs_hbm, idx_hbm, wts_hbm,
                  probs_v, idx_v, wts_v):
    r = rows_per_tile
    wid = lax.axis_index("s") * 2 + lax.axis_index("c")
    base = wid * r
    pltpu.sync_copy(probs_hbm.at[pl.ds(base * n_exp, r * n_exp)], probs_v)

    lanes = lax.iota(jnp.int32, 16)
    zeros = jnp.zeros((16,), jnp.int32)
    neg = jnp.full((16,), -jnp.inf, jnp.float32)

    def per_group(g, carry):
        rowbase = (g * 16 + lanes) * n_exp

        def per_expert(e, c):
            v1, i1, v2, i2 = c
            ev = jnp.full((16,), e, jnp.int32)
            v = plsc.load_gather(probs_v, [rowbase + e])
            gt1 = v > v1
            gt2 = v > v2
            v2n = jnp.where(gt1, v1, jnp.where(gt2, v, v2))
            i2n = jnp.where(gt1, i1, jnp.where(gt2, ev, i2))
            v1n = jnp.where(gt1, v, v1)
            i1n = jnp.where(gt1, ev, i1)
            return v1n, i1n, v2n, i2n

        v1, i1, v2, i2 = lax.fori_loop(
            0, n_exp, per_expert, (neg, zeros, neg, zeros))
        denom = jnp.maximum(v1 + v2, 1e-9)
        pair = (g * 16 + lanes) * 2
        plsc.store_scatter(idx_v, [pair], i1)
        plsc.store_scatter(idx_v, [pair + 1], i2)
        plsc.store_scatter(wts_v, [pair], v1 / denom)
        plsc.store_scatter(wts_v, [pair + 1], v2 / denom)
        return carry

    lax.fori_loop(0, r // 16, per_group, 0)
    pltpu.sync_copy(idx_v, idx_hbm.at[pl.ds(base * 2, r * 2)])
    pltpu.sync_copy(wts_v, wts_hbm.at[pl.ds(base * 2, r * 2)])


def _sc_top2(probs):
    n_tok, n_exp = probs.shape
    r = n_tok // _N_WORKERS
    mesh = plsc.VectorSubcoreMesh(core_axis_name="c", subcore_axis_name="s")
    idx, wts = pl.kernel(
        functools.partial(_sc_top2_body, r, n_exp),
        out_type=[
            jax.ShapeDtypeStruct((n_tok * 2,), jnp.int32),
            jax.ShapeDtypeStruct((n_tok * 2,), jnp.float32),
        ],
        mesh=mesh,
        scratch_types=[
            pltpu.VMEM((r * n_exp,), jnp.float32),
            pltpu.VMEM((r * 2,), jnp.int32),
            pltpu.VMEM((r * 2,), jnp.float32),
        ],
        compiler_params=pltpu.CompilerParams(needs_layout_passes=False),
    )(probs.reshape(-1))
    return idx.reshape(n_tok, 2), wts.reshape(n_tok, 2)


@jax.jit
def kernel(x, W, b):
    probs = _tc_probs(x, W, b)
    idx, wts = _sc_top2(probs)
    return probs, idx, wts


# speedup vs baseline: 1.0982x; 1.0982x over previous
import functools

import jax
import jax.numpy as jnp
from jax import lax
from jax.experimental import pallas as pl
from jax.experimental.pallas import tpu as pltpu
from jax.experimental.pallas import tpu_sc as plsc

_TOK_BLOCK = 2048
_N_WORKERS = 32


def _router_probs_kernel(x_ref, w_ref, b_ref, probs_ref):
    x = x_ref[...]
    logits = jax.lax.dot_general(
        x, w_ref[...], (((1,), (1,)), ((), ())),
        preferred_element_type=jnp.float32,
    )
    logits = logits + b_ref[...]
    m = jnp.max(logits, axis=1, keepdims=True)
    e = jnp.exp(logits - m)
    z = jnp.sum(e, axis=1, keepdims=True)
    probs_ref[...] = e / z


def _tc_probs(x, W, b):
    n_tok, d_model = x.shape
    n_exp = W.shape[0]
    t = _TOK_BLOCK
    return pl.pallas_call(
        _router_probs_kernel,
        grid=(n_tok // t,),
        in_specs=[
            pl.BlockSpec((t, d_model), lambda i: (i, 0)),
            pl.BlockSpec((n_exp, d_model), lambda i: (0, 0)),
            pl.BlockSpec((1, n_exp), lambda i: (0, 0)),
        ],
        out_specs=pl.BlockSpec((t, n_exp), lambda i: (i, 0)),
        out_shape=jax.ShapeDtypeStruct((n_tok, n_exp), jnp.float32),
        compiler_params=pltpu.CompilerParams(
            dimension_semantics=("parallel",),
        ),
    )(x, W.reshape(n_exp, d_model), b.reshape(1, n_exp))


def _sc_top2_body(rows_per_tile, n_exp, probs_hbm, idx_hbm, wts_hbm,
                  probs_v, idx_v, wts_v):
    r = rows_per_tile
    wid = lax.axis_index("s") * 2 + lax.axis_index("c")
    base = wid * r
    pltpu.sync_copy(probs_hbm.at[pl.ds(base * n_exp, r * n_exp)], probs_v)

    lanes = lax.iota(jnp.int32, 16)
    zeros = jnp.zeros((16,), jnp.int32)
    neg = jnp.full((16,), -jnp.inf, jnp.float32)

    def per_group(g, carry):
        rowbase = (g * 16 + lanes) * n_exp

        v1, i1, v2, i2 = neg, zeros, neg, zeros
        for e in range(n_exp):
            ev = jnp.full((16,), e, jnp.int32)
            v = plsc.load_gather(probs_v, [rowbase + e])
            gt1 = v > v1
            gt2 = v > v2
            v2n = jnp.where(gt1, v1, jnp.where(gt2, v, v2))
            i2n = jnp.where(gt1, i1, jnp.where(gt2, ev, i2))
            v1n = jnp.where(gt1, v, v1)
            i1n = jnp.where(gt1, ev, i1)
            v1, i1, v2, i2 = v1n, i1n, v2n, i2n
        denom = jnp.maximum(v1 + v2, 1e-9)
        pair = (g * 16 + lanes) * 2
        plsc.store_scatter(idx_v, [pair], i1)
        plsc.store_scatter(idx_v, [pair + 1], i2)
        plsc.store_scatter(wts_v, [pair], v1 / denom)
        plsc.store_scatter(wts_v, [pair + 1], v2 / denom)
        return carry

    lax.fori_loop(0, r // 16, per_group, 0)
    pltpu.sync_copy(idx_v, idx_hbm.at[pl.ds(base * 2, r * 2)])
    pltpu.sync_copy(wts_v, wts_hbm.at[pl.ds(base * 2, r * 2)])


def _sc_top2(probs):
    n_tok, n_exp = probs.shape
    r = n_tok // _N_WORKERS
    mesh = plsc.VectorSubcoreMesh(core_axis_name="c", subcore_axis_name="s")
    idx, wts = pl.kernel(
        functools.partial(_sc_top2_body, r, n_exp),
        out_type=[
            jax.ShapeDtypeStruct((n_tok * 2,), jnp.int32),
            jax.ShapeDtypeStruct((n_tok * 2,), jnp.float32),
        ],
        mesh=mesh,
        scratch_types=[
            pltpu.VMEM((r * n_exp,), jnp.float32),
            pltpu.VMEM((r * 2,), jnp.int32),
            pltpu.VMEM((r * 2,), jnp.float32),
        ],
        compiler_params=pltpu.CompilerParams(needs_layout_passes=False),
    )(probs.reshape(-1))
    return idx.reshape(n_tok, 2), wts.reshape(n_tok, 2)


@jax.jit
def kernel(x, W, b):
    probs = _tc_probs(x, W, b)
    idx, wts = _sc_top2(probs)
    return probs, idx, wts


# speedup vs baseline: 1.8520x; 1.6864x over previous
import functools

import jax
import jax.numpy as jnp
from jax.experimental import pallas as pl
from jax.experimental.pallas import tpu as pltpu

_TOK_BLOCK = 2048


def _router_kernel(x_ref, w_ref, b_ref, probs_ref, idx_ref, wts_ref):
    x = x_ref[...]
    logits = jax.lax.dot_general(
        x, w_ref[...], (((1,), (1,)), ((), ())),
        preferred_element_type=jnp.float32,
    )
    logits = logits + b_ref[...]

    cols = jax.lax.broadcasted_iota(jnp.int32, logits.shape, 1)
    n = logits.shape[1]
    v1 = jnp.max(logits, axis=1, keepdims=True)
    i1 = jnp.min(jnp.where(logits == v1, cols, n), axis=1, keepdims=True)
    masked = jnp.where(cols == i1, -jnp.inf, logits)
    v2 = jnp.max(masked, axis=1, keepdims=True)
    i2 = jnp.min(jnp.where(masked == v2, cols, n), axis=1, keepdims=True)
    w1 = 1.0 / (1.0 + jnp.exp(v2 - v1))
    wts_ref[...] = jnp.concatenate([w1, 1.0 - w1], axis=1)
    idx_ref[...] = jnp.concatenate([i1, i2], axis=1)

    e = jnp.exp(logits - v1)
    z = jnp.sum(e, axis=1, keepdims=True)
    probs_ref[...] = e / z


@jax.jit
def kernel(x, W, b):
    n_tok, d_model = x.shape
    n_exp = W.shape[0]
    t = _TOK_BLOCK
    grid = (n_tok // t,)
    probs, idx, wts = pl.pallas_call(
        _router_kernel,
        grid=grid,
        in_specs=[
            pl.BlockSpec((t, d_model), lambda i: (i, 0)),
            pl.BlockSpec((n_exp, d_model), lambda i: (0, 0)),
            pl.BlockSpec((1, n_exp), lambda i: (0, 0)),
        ],
        out_specs=[
            pl.BlockSpec((t, n_exp), lambda i: (i, 0)),
            pl.BlockSpec((t, 2), lambda i: (i, 0)),
            pl.BlockSpec((t, 2), lambda i: (i, 0)),
        ],
        out_shape=[
            jax.ShapeDtypeStruct((n_tok, n_exp), jnp.float32),
            jax.ShapeDtypeStruct((n_tok, 2), jnp.int32),
            jax.ShapeDtypeStruct((n_tok, 2), jnp.float32),
        ],
        compiler_params=pltpu.CompilerParams(
            dimension_semantics=("parallel",),
        ),
    )(x, W.reshape(n_exp, d_model), b.reshape(1, n_exp))
    return probs, idx, wts
